# baseline (speedup 1.0000x reference)
import functools

import jax
import jax.numpy as jnp
import numpy as np
from jax import lax
from jax.experimental import pallas as pl
from jax.experimental.pallas import tpu as pltpu
from jax.experimental.pallas import tpu_sc as plsc

B = 16384
D = 64
BN_EPS = 1e-5

NC = 2
NS = 16
NW = NC * NS
BPW = B // NW
CHUNK = 32


@functools.lru_cache(maxsize=None)
def _make_sc_gather(nu, nm):
    mesh = plsc.VectorSubcoreMesh(core_axis_name="c", subcore_axis_name="s")

    @functools.partial(
        pl.kernel,
        mesh=mesh,
        compiler_params=pltpu.CompilerParams(needs_layout_passes=False),
        out_type=[
            jax.ShapeDtypeStruct((NW, BPW, D), jnp.float32),
            jax.ShapeDtypeStruct((NW, BPW, D), jnp.float32),
        ],
        scratch_types=[
            pltpu.VMEM((BPW,), jnp.int32),
            pltpu.VMEM((BPW,), jnp.int32),
            pltpu.VMEM((CHUNK, 8, 2 * D), jnp.float32),
            pltpu.VMEM((BPW, D), jnp.float32),
            pltpu.SemaphoreType.DMA,
            pltpu.SemaphoreType.DMA,
        ],
    )
    def _sc_gather(users_hbm, movies_hbm, ut_hbm, mt_hbm, ue_hbm, me_hbm,
                   idx_u, idx_m, rows_g, out_rows, sem0, sem1):
        wid = lax.axis_index("s") * NC + lax.axis_index("c")
        base = pl.multiple_of(wid * BPW, BPW)
        pltpu.sync_copy(users_hbm.at[pl.ds(base, BPW)], idx_u)
        pltpu.sync_copy(movies_hbm.at[pl.ds(base, BPW)], idx_m)
        sems = (sem0, sem1)
        NG = BPW // 16

        def one_table(idx_ref, tbl_hbm, out_hbm):
            def fire(gi, half):
                iv = idx_ref[pl.ds(gi * 16, 16)]
                gv = lax.shift_right_logical(iv, 4)
                for j in range(16):
                    pltpu.async_copy(tbl_hbm.at[gv[j]],
                                     rows_g.at[half * 16 + j], sems[half])

            def drain_process(gi, half):
                base = gi * 16
                iv = idx_ref[pl.ds(base, 16)]
                sv = lax.bitwise_and(lax.shift_right_logical(iv, 1), 7)
                hv = lax.bitwise_and(iv, 1) * D
                for j in range(16):
                    slot = half * 16 + j
                    pltpu.make_async_copy(tbl_hbm.at[0],
                                          rows_g.at[slot], sems[half]).wait()
                    s = sv[j]
                    ho = hv[j]
                    for c in range(D // 16):
                        off = pl.multiple_of(ho + c * 16, 16)
                        out_rows[base + j, pl.ds(c * 16, 16)] = (
                            rows_g[slot, s, pl.ds(off, 16)])

            fire(0, 0)

            def body(p, _):
                gi_a = 2 * p + 1

                @pl.when(gi_a < NG)
                def _():
                    fire(gi_a, 1)

                drain_process(2 * p, 0)
                gi_b = 2 * p + 2

                @pl.when(gi_b < NG)
                def _():
                    fire(gi_b, 0)

                drain_process(2 * p + 1, 1)
                return 0

            lax.fori_loop(0, NG // 2, body, 0)
            pltpu.sync_copy(out_rows, out_hbm.at[wid])

        one_table(idx_u, ut_hbm, ue_hbm)
        one_table(idx_m, mt_hbm, me_hbm)

    return _sc_gather


def _mlp_body(ue_ref, me_ref, w1_ref, c1_ref, w2_ref, c2_ref, w3_ref, c3_ref,
              out_ref):
    w1 = w1_ref[...]
    nt = (((1,), (1,)), ((), ()))
    h = lax.dot_general(ue_ref[0], w1[:, :D], nt,
                        preferred_element_type=jnp.float32)
    h += lax.dot_general(me_ref[0], w1[:, D:], nt,
                         preferred_element_type=jnp.float32)
    h = jnp.maximum(h + c1_ref[...], 0.0)
    h = lax.dot_general(h, w2_ref[...], nt, preferred_element_type=jnp.float32)
    h = jnp.maximum(h + c2_ref[...], 0.0)
    o = jnp.sum(h * w3_ref[...], axis=1, keepdims=True) + c3_ref[...]
    out_ref[...] = 1.0 / (1.0 + jnp.exp(-o))


def kernel(users, movies, user_table, movie_table,
           W1, b1, g1, be1, W2, b2, g2, be2, W3, b3):
    u = users.astype(jnp.int32)
    m = movies.astype(jnp.int32)
    ut3 = user_table.reshape(user_table.shape[0] // 16, 8, 2 * D)
    mt3 = movie_table.reshape(movie_table.shape[0] // 16, 8, 2 * D)
    ue3, me3 = _make_sc_gather(ut3.shape[0], mt3.shape[0])(u, m, ut3, mt3)

    s = np.float32(1.0 / np.sqrt(1.0 + BN_EPS))
    w1 = W1 * (g1 * s)[:, None]
    c1 = (b1 * g1 * s + be1).reshape(1, 32)
    w2 = W2 * (g2 * s)[:, None]
    c2 = (b2 * g2 * s + be2).reshape(1, 16)
    w3 = W3.reshape(1, 16)
    c3 = b3.reshape(1, 1)

    out = pl.pallas_call(
        _mlp_body,
        grid=(NW,),
        in_specs=[
            pl.BlockSpec((1, BPW, D), lambda w: (w, 0, 0)),
            pl.BlockSpec((1, BPW, D), lambda w: (w, 0, 0)),
            pl.BlockSpec((32, 128), lambda w: (0, 0)),
            pl.BlockSpec((1, 32), lambda w: (0, 0)),
            pl.BlockSpec((16, 32), lambda w: (0, 0)),
            pl.BlockSpec((1, 16), lambda w: (0, 0)),
            pl.BlockSpec((1, 16), lambda w: (0, 0)),
            pl.BlockSpec((1, 1), lambda w: (0, 0)),
        ],
        out_specs=pl.BlockSpec((BPW, 1), lambda w: (w, 0)),
        out_shape=jax.ShapeDtypeStruct((B, 1), jnp.float32),
    )(ue3, me3, w1, c1, w2, c2, w3, c3)
    return out


# speedup vs baseline: 2.0969x; 2.0969x over previous
import functools

import jax
import jax.numpy as jnp
import numpy as np
from jax import lax
from jax.experimental import pallas as pl
from jax.experimental.pallas import tpu as pltpu
from jax.experimental.pallas import tpu_sc as plsc

B = 16384
D = 64
BN_EPS = 1e-5

NC = 2
NS = 16
NW = NC * NS
BPW = B // NW
CHUNK = 32


@functools.lru_cache(maxsize=None)
def _make_sc_gather(nu, nm):
    mesh = plsc.VectorSubcoreMesh(core_axis_name="c", subcore_axis_name="s")

    @functools.partial(
        pl.kernel,
        mesh=mesh,
        compiler_params=pltpu.CompilerParams(needs_layout_passes=False),
        out_type=[
            jax.ShapeDtypeStruct((NW, BPW, D), jnp.float32),
            jax.ShapeDtypeStruct((NW, BPW, D), jnp.float32),
        ],
        scratch_types=[
            pltpu.VMEM((BPW,), jnp.int32),
            pltpu.VMEM((BPW,), jnp.int32),
            pltpu.VMEM((CHUNK, 8, D), jnp.float32),
            pltpu.VMEM((BPW, D), jnp.float32),
            pltpu.SemaphoreType.DMA,
            pltpu.SemaphoreType.DMA,
        ],
    )
    def _sc_gather(users_hbm, movies_hbm, ut_hbm, mt_hbm, ue_hbm, me_hbm,
                   idx_u, idx_m, rows_g, out_rows, sem0, sem1):
        wid = lax.axis_index("s") * NC + lax.axis_index("c")
        base = pl.multiple_of(wid * BPW, BPW)
        pltpu.sync_copy(users_hbm.at[pl.ds(base, BPW)], idx_u)
        pltpu.sync_copy(movies_hbm.at[pl.ds(base, BPW)], idx_m)
        sems = (sem0, sem1)
        NG = BPW // 16

        def one_table(idx_ref, tbl_hbm, out_hbm):
            def fire(gi, half):
                iv = idx_ref[pl.ds(gi * 16, 16)]
                gv = lax.shift_right_logical(iv, 3)
                for j in range(16):
                    pltpu.async_copy(tbl_hbm.at[gv[j]],
                                     rows_g.at[half * 16 + j], sems[half])

            def drain_process(gi, half):
                base = gi * 16
                iv = idx_ref[pl.ds(base, 16)]
                sv = lax.bitwise_and(iv, 7)
                for j in range(16):
                    slot = half * 16 + j
                    pltpu.make_async_copy(tbl_hbm.at[0],
                                          rows_g.at[slot], sems[half]).wait()
                    s = sv[j]
                    for c in range(D // 16):
                        out_rows[base + j, pl.ds(c * 16, 16)] = (
                            rows_g[slot, s, pl.ds(c * 16, 16)])

            fire(0, 0)

            def body(p, _):
                gi_a = 2 * p + 1

                @pl.when(gi_a < NG)
                def _():
                    fire(gi_a, 1)

                drain_process(2 * p, 0)
                gi_b = 2 * p + 2

                @pl.when(gi_b < NG)
                def _():
                    fire(gi_b, 0)

                drain_process(2 * p + 1, 1)
                return 0

            lax.fori_loop(0, NG // 2, body, 0)
            pltpu.sync_copy(out_rows, out_hbm.at[wid])

        one_table(idx_u, ut_hbm, ue_hbm)
        one_table(idx_m, mt_hbm, me_hbm)

    return _sc_gather


def _mlp_body(ue_ref, me_ref, w1_ref, c1_ref, w2_ref, c2_ref, w3_ref, c3_ref,
              out_ref):
    w1 = w1_ref[...]
    nt = (((1,), (1,)), ((), ()))
    h = lax.dot_general(ue_ref[0], w1[:, :D], nt,
                        preferred_element_type=jnp.float32)
    h += lax.dot_general(me_ref[0], w1[:, D:], nt,
                         preferred_element_type=jnp.float32)
    h = jnp.maximum(h + c1_ref[...], 0.0)
    h = lax.dot_general(h, w2_ref[...], nt, preferred_element_type=jnp.float32)
    h = jnp.maximum(h + c2_ref[...], 0.0)
    o = jnp.sum(h * w3_ref[...], axis=1, keepdims=True) + c3_ref[...]
    out_ref[...] = 1.0 / (1.0 + jnp.exp(-o))


def kernel(users, movies, user_table, movie_table,
           W1, b1, g1, be1, W2, b2, g2, be2, W3, b3):
    u = users.astype(jnp.int32)
    m = movies.astype(jnp.int32)
    ut3 = user_table.reshape(user_table.shape[0] // 8, 8, D)
    mt3 = movie_table.reshape(movie_table.shape[0] // 8, 8, D)
    ue3, me3 = _make_sc_gather(ut3.shape[0], mt3.shape[0])(u, m, ut3, mt3)

    s = np.float32(1.0 / np.sqrt(1.0 + BN_EPS))
    w1 = W1 * (g1 * s)[:, None]
    c1 = (b1 * g1 * s + be1).reshape(1, 32)
    w2 = W2 * (g2 * s)[:, None]
    c2 = (b2 * g2 * s + be2).reshape(1, 16)
    w3 = W3.reshape(1, 16)
    c3 = b3.reshape(1, 1)

    out = pl.pallas_call(
        _mlp_body,
        grid=(NW,),
        in_specs=[
            pl.BlockSpec((1, BPW, D), lambda w: (w, 0, 0)),
            pl.BlockSpec((1, BPW, D), lambda w: (w, 0, 0)),
            pl.BlockSpec((32, 128), lambda w: (0, 0)),
            pl.BlockSpec((1, 32), lambda w: (0, 0)),
            pl.BlockSpec((16, 32), lambda w: (0, 0)),
            pl.BlockSpec((1, 16), lambda w: (0, 0)),
            pl.BlockSpec((1, 16), lambda w: (0, 0)),
            pl.BlockSpec((1, 1), lambda w: (0, 0)),
        ],
        out_specs=pl.BlockSpec((BPW, 1), lambda w: (w, 0)),
        out_shape=jax.ShapeDtypeStruct((B, 1), jnp.float32),
    )(ue3, me3, w1, c1, w2, c2, w3, c3)
    return out
